# baseline (speedup 1.0000x reference)
import functools

import jax
import jax.numpy as jnp
from jax import lax
from jax.experimental import pallas as pl
from jax.experimental.pallas import tpu as pltpu
from jax.experimental.pallas import tpu_sc as plsc

_LANES = 16


def _seq_chunks(S):
    chunks = []
    off = 0
    while off < S:
        c = min(128, S - off)
        chunks.append((off, c))
        off += c
    return chunks


def _transpose_pack(tabT, V, D, H):
    BK = 128

    def body(x1_ref, x2_ref, o_ref):
        o_ref[...] = jnp.concatenate([x1_ref[...].T, x2_ref[...].T], axis=1)

    return pl.pallas_call(
        body,
        grid=(H // BK,),
        in_specs=[
            pl.BlockSpec((D, BK), lambda i: (0, i)),
            pl.BlockSpec((D, BK), lambda i: (0, i + H // BK)),
        ],
        out_specs=pl.BlockSpec((BK, 2 * D), lambda i: (i, 0)),
        out_shape=jax.ShapeDtypeStruct((H, 2 * D), jnp.float32),
    )(tabT, tabT)


def _pooled_sc(indices_flat, tab_lin, H, B, S):
    V2, D = tab_lin.shape
    info = plsc.get_sparse_core_info()
    NC, NS = info.num_cores, info.num_subcores
    NW = NC * NS
    assert B % NW == 0
    rows_per_w = B // NW
    chunks = _seq_chunks(S)
    nd = D // _LANES
    assert (rows_per_w * S) % _LANES == 0 and S % 4 == 0
    nbuf = 2
    assert rows_per_w % nbuf == 0
    sub = 2 * H - 1

    mesh = plsc.VectorSubcoreMesh(core_axis_name="c", subcore_axis_name="s")

    @functools.partial(
        pl.kernel,
        mesh=mesh,
        out_type=jax.ShapeDtypeStruct((B, D), jnp.float32),
        scratch_types=[
            pltpu.VMEM((rows_per_w * S,), jnp.int32),
            pltpu.VMEM((nbuf, S, D), jnp.float32),
            pltpu.VMEM((rows_per_w, D), jnp.float32),
            pltpu.SemaphoreType.DMA,
            pltpu.SemaphoreType.DMA,
        ],
        compiler_params=pltpu.CompilerParams(use_tc_tiling_on_sc=False),
    )
    def k(idx_hbm, tab_hbm, out_hbm, idx_v, buf, pooled_v, sem0, sem1):
        wid = lax.axis_index("s") * NC + lax.axis_index("c")
        base = wid * rows_per_w
        sems = (sem0, sem1)
        pltpu.sync_copy(idx_hbm.at[pl.ds(base * S, rows_per_w * S)], idx_v)

        flat = rows_per_w * S

        def remap(g, carry):
            r = idx_v[pl.ds(g * _LANES, _LANES)]
            m = r + r - jnp.where(r >= H, jnp.int32(sub), jnp.int32(0))
            idx_v[pl.ds(g * _LANES, _LANES)] = m
            return carry

        lax.fori_loop(0, flat // _LANES, remap, 0)

        def issue(i, slot, sem):
            for off, c in chunks:
                pltpu.async_copy(
                    tab_hbm.at[idx_v.at[pl.ds(i * S + off, c)]],
                    buf.at[slot, pl.ds(off, c)],
                    sem,
                )

        def wait(i, slot, sem):
            for off, c in chunks:
                pltpu.make_async_copy(
                    tab_hbm.at[idx_v.at[pl.ds(i * S + off, c)]],
                    buf.at[slot, pl.ds(off, c)],
                    sem,
                ).wait()

        issue(0, 0, sems[0])

        def outer(t, carry):
            for s in range(nbuf):
                i = t * nbuf + s
                ns = (s + 1) % nbuf

                @pl.when(i + 1 < rows_per_w)
                def _():
                    issue(i + 1, ns, sems[ns])

                wait(i, s, sems[s])

                accs = tuple(
                    buf[s, 0, pl.ds(d * _LANES, _LANES)] for d in range(nd)
                )
                for r in (1, 2, 3):
                    accs = tuple(
                        jnp.maximum(a, buf[s, r, pl.ds(d * _LANES, _LANES)])
                        for d, a in enumerate(accs)
                    )

                def red4(kk, accs, s=s):
                    jbase = kk * 4
                    for r in range(4):
                        accs = tuple(
                            jnp.maximum(
                                a, buf[s, jbase + r, pl.ds(d * _LANES, _LANES)]
                            )
                            for d, a in enumerate(accs)
                        )
                    return accs

                accs = lax.fori_loop(1, S // 4, red4, accs)
                for d, a in enumerate(accs):
                    pooled_v[i, pl.ds(d * _LANES, _LANES)] = a
            return carry

        lax.fori_loop(0, rows_per_w // nbuf, outer, 0)
        pltpu.sync_copy(pooled_v, out_hbm.at[pl.ds(base, rows_per_w)])

    return k(indices_flat, tab_lin)


def _linear_tc(pooled, W, b):
    B, D = pooled.shape
    O = W.shape[0]
    blk = 512

    def mm(p_ref, w_ref, b_ref, o_ref):
        o_ref[...] = (
            lax.dot_general(
                p_ref[...],
                w_ref[...],
                dimension_numbers=(((1,), (1,)), ((), ())),
                preferred_element_type=jnp.float32,
            )
            + b_ref[...]
        )

    return pl.pallas_call(
        mm,
        grid=(B // blk,),
        in_specs=[
            pl.BlockSpec((blk, D), lambda i: (i, 0)),
            pl.BlockSpec((O, D), lambda i: (0, 0)),
            pl.BlockSpec((1, O), lambda i: (0, 0)),
        ],
        out_specs=pl.BlockSpec((blk, O), lambda i: (i, 0)),
        out_shape=jax.ShapeDtypeStruct((B, O), jnp.float32),
    )(pooled, W, b.reshape(1, O))


def kernel(indices, emb_table, W, b):
    V, D = emb_table.shape
    H = 500096
    assert V == 1000000 and D == 64
    P = _transpose_pack(emb_table.T, V, D, H)
    tab_lin = P.reshape(2 * H, D)
    B, S = indices.shape
    pooled = _pooled_sc(indices.reshape(B * S), tab_lin, H, B, S)
    return _linear_tc(pooled, W, b)


# speedup vs baseline: 4.1354x; 4.1354x over previous
import functools

import jax
import jax.numpy as jnp
from jax import lax
from jax.experimental import pallas as pl
from jax.experimental.pallas import tpu as pltpu
from jax.experimental.pallas import tpu_sc as plsc

_LANES = 16


def _seq_chunks(S):
    chunks = []
    off = 0
    while off < S:
        c = min(128, S - off)
        chunks.append((off, c))
        off += c
    return chunks


def _transpose_pack(tabT, V, D, H):
    TR = 128
    BK = 1024
    SUB = BK // TR
    eye = jnp.eye(TR, dtype=jnp.float32)

    def body(e_ref, x1_ref, x2_ref, o_ref):
        e = e_ref[...]
        dn = (((1,), (1,)), ((), ()))
        for m in range(SUB):
            sl = pl.ds(m * TR, TR)
            t1 = lax.dot_general(
                e, x1_ref[:, sl], dn, preferred_element_type=jnp.float32
            )
            t2 = lax.dot_general(
                e, x2_ref[:, sl], dn, preferred_element_type=jnp.float32
            )
            o_ref[sl, :] = jnp.concatenate([t1, t2], axis=1)

    max_blk = (V - 1) // BK

    return pl.pallas_call(
        body,
        grid=(H // BK,),
        in_specs=[
            pl.BlockSpec((TR, TR), lambda i: (0, 0)),
            pl.BlockSpec((D, BK), lambda i: (0, i)),
            pl.BlockSpec((D, BK), lambda i: (0, jnp.minimum(i + H // BK, max_blk))),
        ],
        out_specs=pl.BlockSpec((BK, 2 * D), lambda i: (i, 0)),
        out_shape=jax.ShapeDtypeStruct((H, 2 * D), jnp.float32),
    )(eye, tabT, tabT)


def _pooled_sc(indices_flat, tab_lin, H, B, S):
    V2, D = tab_lin.shape
    info = plsc.get_sparse_core_info()
    NC, NS = info.num_cores, info.num_subcores
    NW = NC * NS
    assert B % NW == 0
    rows_per_w = B // NW
    chunks = _seq_chunks(S)
    nd = D // _LANES
    assert (rows_per_w * S) % _LANES == 0 and S % 4 == 0
    nbuf = 2
    assert rows_per_w % nbuf == 0
    sub = 2 * H - 1

    mesh = plsc.VectorSubcoreMesh(core_axis_name="c", subcore_axis_name="s")

    @functools.partial(
        pl.kernel,
        mesh=mesh,
        out_type=jax.ShapeDtypeStruct((B, D), jnp.float32),
        scratch_types=[
            pltpu.VMEM((rows_per_w * S,), jnp.int32),
            pltpu.VMEM((nbuf, S, D), jnp.float32),
            pltpu.VMEM((rows_per_w, D), jnp.float32),
            pltpu.SemaphoreType.DMA,
            pltpu.SemaphoreType.DMA,
        ],
        compiler_params=pltpu.CompilerParams(use_tc_tiling_on_sc=False),
    )
    def k(idx_hbm, tab_hbm, out_hbm, idx_v, buf, pooled_v, sem0, sem1):
        wid = lax.axis_index("s") * NC + lax.axis_index("c")
        base = wid * rows_per_w
        sems = (sem0, sem1)
        pltpu.sync_copy(idx_hbm.at[pl.ds(base * S, rows_per_w * S)], idx_v)

        flat = rows_per_w * S

        def remap(g, carry):
            r = idx_v[pl.ds(g * _LANES, _LANES)]
            m = r + r - jnp.where(r >= H, jnp.int32(sub), jnp.int32(0))
            idx_v[pl.ds(g * _LANES, _LANES)] = m
            return carry

        lax.fori_loop(0, flat // _LANES, remap, 0)

        def issue(i, slot, sem):
            for off, c in chunks:
                pltpu.async_copy(
                    tab_hbm.at[idx_v.at[pl.ds(i * S + off, c)]],
                    buf.at[slot, pl.ds(off, c)],
                    sem,
                )

        def wait(i, slot, sem):
            for off, c in chunks:
                pltpu.make_async_copy(
                    tab_hbm.at[idx_v.at[pl.ds(i * S + off, c)]],
                    buf.at[slot, pl.ds(off, c)],
                    sem,
                ).wait()

        issue(0, 0, sems[0])

        def outer(t, carry):
            for s in range(nbuf):
                i = t * nbuf + s
                ns = (s + 1) % nbuf

                @pl.when(i + 1 < rows_per_w)
                def _():
                    issue(i + 1, ns, sems[ns])

                wait(i, s, sems[s])

                accs = tuple(
                    buf[s, 0, pl.ds(d * _LANES, _LANES)] for d in range(nd)
                )
                for r in (1, 2, 3):
                    accs = tuple(
                        jnp.maximum(a, buf[s, r, pl.ds(d * _LANES, _LANES)])
                        for d, a in enumerate(accs)
                    )

                def red4(kk, accs, s=s):
                    jbase = kk * 4
                    for r in range(4):
                        accs = tuple(
                            jnp.maximum(
                                a, buf[s, jbase + r, pl.ds(d * _LANES, _LANES)]
                            )
                            for d, a in enumerate(accs)
                        )
                    return accs

                accs = lax.fori_loop(1, S // 4, red4, accs)
                for d, a in enumerate(accs):
                    pooled_v[i, pl.ds(d * _LANES, _LANES)] = a
            return carry

        lax.fori_loop(0, rows_per_w // nbuf, outer, 0)
        pltpu.sync_copy(pooled_v, out_hbm.at[pl.ds(base, rows_per_w)])

    return k(indices_flat, tab_lin)


def _linear_tc(pooled, W, b):
    B, D = pooled.shape
    O = W.shape[0]
    blk = 512

    def mm(p_ref, w_ref, b_ref, o_ref):
        o_ref[...] = (
            lax.dot_general(
                p_ref[...],
                w_ref[...],
                dimension_numbers=(((1,), (1,)), ((), ())),
                preferred_element_type=jnp.float32,
            )
            + b_ref[...]
        )

    return pl.pallas_call(
        mm,
        grid=(B // blk,),
        in_specs=[
            pl.BlockSpec((blk, D), lambda i: (i, 0)),
            pl.BlockSpec((O, D), lambda i: (0, 0)),
            pl.BlockSpec((1, O), lambda i: (0, 0)),
        ],
        out_specs=pl.BlockSpec((blk, O), lambda i: (i, 0)),
        out_shape=jax.ShapeDtypeStruct((B, O), jnp.float32),
    )(pooled, W, b.reshape(1, O))


def kernel(indices, emb_table, W, b):
    V, D = emb_table.shape
    H = 500736
    assert V == 1000000 and D == 64
    P = _transpose_pack(emb_table.T, V, D, H)
    tab_lin = P.reshape(2 * H, D)
    B, S = indices.shape
    pooled = _pooled_sc(indices.reshape(B * S), tab_lin, H, B, S)
    return _linear_tc(pooled, W, b)


# speedup vs baseline: 5.4098x; 1.3082x over previous
import functools

import jax
import jax.numpy as jnp
from jax import lax
from jax.experimental import pallas as pl
from jax.experimental.pallas import tpu as pltpu
from jax.experimental.pallas import tpu_sc as plsc

_LANES = 16


def _seq_chunks(S):
    chunks = []
    off = 0
    while off < S:
        c = min(128, S - off)
        chunks.append((off, c))
        off += c
    return chunks


def _transpose_pack(tabT, V, D, H):
    TR = 128
    BK = 2048
    SUB = BK // TR
    eye = jnp.eye(TR, dtype=jnp.float32)

    def body(e_ref, x1_ref, x2_ref, o_ref):
        e = e_ref[...]
        dn = (((1,), (1,)), ((), ()))
        for m in range(SUB):
            sl = pl.ds(m * TR, TR)
            t1 = lax.dot_general(
                e, x1_ref[:, sl], dn, preferred_element_type=jnp.float32
            )
            t2 = lax.dot_general(
                e, x2_ref[:, sl], dn, preferred_element_type=jnp.float32
            )
            o_ref[sl, :] = jnp.concatenate([t1, t2], axis=1)

    max_blk = (V - 1) // BK

    return pl.pallas_call(
        body,
        grid=(H // BK,),
        in_specs=[
            pl.BlockSpec((TR, TR), lambda i: (0, 0)),
            pl.BlockSpec((D, BK), lambda i: (0, i)),
            pl.BlockSpec((D, BK), lambda i: (0, jnp.minimum(i + H // BK, max_blk))),
        ],
        out_specs=pl.BlockSpec((BK, 2 * D), lambda i: (i, 0)),
        out_shape=jax.ShapeDtypeStruct((H, 2 * D), jnp.float32),
    )(eye, tabT, tabT)


def _pooled_sc(indices_flat, tab_lin, H, B, S):
    V2, D = tab_lin.shape
    info = plsc.get_sparse_core_info()
    NC, NS = info.num_cores, info.num_subcores
    NW = NC * NS
    assert B % NW == 0
    rows_per_w = B // NW
    chunks = _seq_chunks(S)
    nd = D // _LANES
    assert (rows_per_w * S) % _LANES == 0 and S % 4 == 0
    nbuf = 2
    assert rows_per_w % nbuf == 0
    sub = 2 * H - 1

    mesh = plsc.VectorSubcoreMesh(core_axis_name="c", subcore_axis_name="s")

    @functools.partial(
        pl.kernel,
        mesh=mesh,
        out_type=jax.ShapeDtypeStruct((B, D), jnp.float32),
        scratch_types=[
            pltpu.VMEM((rows_per_w * S,), jnp.int32),
            pltpu.VMEM((nbuf, S, D), jnp.float32),
            pltpu.VMEM((rows_per_w, D), jnp.float32),
            pltpu.SemaphoreType.DMA,
            pltpu.SemaphoreType.DMA,
        ],
        compiler_params=pltpu.CompilerParams(use_tc_tiling_on_sc=False),
    )
    def k(idx_hbm, tab_hbm, out_hbm, idx_v, buf, pooled_v, sem0, sem1):
        wid = lax.axis_index("s") * NC + lax.axis_index("c")
        base = wid * rows_per_w
        sems = (sem0, sem1)
        pltpu.sync_copy(idx_hbm.at[pl.ds(base * S, rows_per_w * S)], idx_v)

        flat = rows_per_w * S

        def remap(g, carry):
            r = idx_v[pl.ds(g * _LANES, _LANES)]
            m = r + r - jnp.where(r >= H, jnp.int32(sub), jnp.int32(0))
            idx_v[pl.ds(g * _LANES, _LANES)] = m
            return carry

        lax.fori_loop(0, flat // _LANES, remap, 0)

        def issue(i, slot, sem):
            for off, c in chunks:
                pltpu.async_copy(
                    tab_hbm.at[idx_v.at[pl.ds(i * S + off, c)]],
                    buf.at[slot, pl.ds(off, c)],
                    sem,
                )

        def wait(i, slot, sem):
            for off, c in chunks:
                pltpu.make_async_copy(
                    tab_hbm.at[idx_v.at[pl.ds(i * S + off, c)]],
                    buf.at[slot, pl.ds(off, c)],
                    sem,
                ).wait()

        issue(0, 0, sems[0])

        def outer(t, carry):
            for s in range(nbuf):
                i = t * nbuf + s
                ns = (s + 1) % nbuf

                @pl.when(i + 1 < rows_per_w)
                def _():
                    issue(i + 1, ns, sems[ns])

                wait(i, s, sems[s])

                accs = tuple(
                    buf[s, 0, pl.ds(d * _LANES, _LANES)] for d in range(nd)
                )
                for r in (1, 2, 3):
                    accs = tuple(
                        jnp.maximum(a, buf[s, r, pl.ds(d * _LANES, _LANES)])
                        for d, a in enumerate(accs)
                    )

                def red4(kk, accs, s=s):
                    jbase = kk * 4
                    for r in range(4):
                        accs = tuple(
                            jnp.maximum(
                                a, buf[s, jbase + r, pl.ds(d * _LANES, _LANES)]
                            )
                            for d, a in enumerate(accs)
                        )
                    return accs

                accs = lax.fori_loop(1, S // 4, red4, accs)
                for d, a in enumerate(accs):
                    pooled_v[i, pl.ds(d * _LANES, _LANES)] = a
            return carry

        lax.fori_loop(0, rows_per_w // nbuf, outer, 0)
        pltpu.sync_copy(pooled_v, out_hbm.at[pl.ds(base, rows_per_w)])

    return k(indices_flat, tab_lin)


def _linear_tc(pooled, W, b):
    B, D = pooled.shape
    O = W.shape[0]
    blk = 512

    def mm(p_ref, w_ref, b_ref, o_ref):
        o_ref[...] = (
            lax.dot_general(
                p_ref[...],
                w_ref[...],
                dimension_numbers=(((1,), (1,)), ((), ())),
                preferred_element_type=jnp.float32,
            )
            + b_ref[...]
        )

    return pl.pallas_call(
        mm,
        grid=(B // blk,),
        in_specs=[
            pl.BlockSpec((blk, D), lambda i: (i, 0)),
            pl.BlockSpec((O, D), lambda i: (0, 0)),
            pl.BlockSpec((1, O), lambda i: (0, 0)),
        ],
        out_specs=pl.BlockSpec((blk, O), lambda i: (i, 0)),
        out_shape=jax.ShapeDtypeStruct((B, O), jnp.float32),
    )(pooled, W, b.reshape(1, O))


def kernel(indices, emb_table, W, b):
    V, D = emb_table.shape
    H = 501760
    assert V == 1000000 and D == 64
    P = _transpose_pack(emb_table.T, V, D, H)
    tab_lin = P.reshape(2 * H, D)
    B, S = indices.shape
    pooled = _pooled_sc(indices.reshape(B * S), tab_lin, H, B, S)
    return _linear_tc(pooled, W, b)


# speedup vs baseline: 7.3146x; 1.3521x over previous
import functools

import jax
import jax.numpy as jnp
from jax import lax
from jax.experimental import pallas as pl
from jax.experimental.pallas import tpu as pltpu
from jax.experimental.pallas import tpu_sc as plsc

_LANES = 16


def _seq_chunks(S):
    chunks = []
    off = 0
    while off < S:
        c = min(128, S - off)
        chunks.append((off, c))
        off += c
    return chunks


def _transpose_pack(tabT, V, D, H):
    TR = 128
    BK = 8192
    SUB = BK // TR
    eye = jnp.eye(TR, dtype=jnp.float32)

    def body(e_ref, x1_ref, x2_ref, o_ref):
        e = e_ref[...]
        dn = (((1,), (1,)), ((), ()))
        for m in range(SUB):
            sl = pl.ds(m * TR, TR)
            t1 = lax.dot_general(
                e, x1_ref[:, sl], dn, preferred_element_type=jnp.float32
            )
            t2 = lax.dot_general(
                e, x2_ref[:, sl], dn, preferred_element_type=jnp.float32
            )
            o_ref[sl, :] = jnp.concatenate([t1, t2], axis=1)

    max_blk = (V - 1) // BK

    return pl.pallas_call(
        body,
        grid=(H // BK,),
        in_specs=[
            pl.BlockSpec((TR, TR), lambda i: (0, 0)),
            pl.BlockSpec((D, BK), lambda i: (0, i)),
            pl.BlockSpec((D, BK), lambda i: (0, jnp.minimum(i + H // BK, max_blk))),
        ],
        out_specs=pl.BlockSpec((BK, 2 * D), lambda i: (i, 0)),
        out_shape=jax.ShapeDtypeStruct((H, 2 * D), jnp.float32),
    )(eye, tabT, tabT)


def _pooled_sc(indices_flat, tab_lin, H, B, S):
    V2, D = tab_lin.shape
    info = plsc.get_sparse_core_info()
    NC, NS = info.num_cores, info.num_subcores
    NW = NC * NS
    assert B % NW == 0
    rows_per_w = B // NW
    chunks = _seq_chunks(S)
    nd = D // _LANES
    assert (rows_per_w * S) % _LANES == 0 and S % 4 == 0
    nbuf = 2
    assert rows_per_w % nbuf == 0
    sub = 2 * H - 1

    mesh = plsc.VectorSubcoreMesh(core_axis_name="c", subcore_axis_name="s")

    @functools.partial(
        pl.kernel,
        mesh=mesh,
        out_type=jax.ShapeDtypeStruct((B, D), jnp.float32),
        scratch_types=[
            pltpu.VMEM((rows_per_w * S,), jnp.int32),
            pltpu.VMEM((nbuf, S, D), jnp.float32),
            pltpu.VMEM((rows_per_w, D), jnp.float32),
            pltpu.SemaphoreType.DMA,
            pltpu.SemaphoreType.DMA,
        ],
        compiler_params=pltpu.CompilerParams(use_tc_tiling_on_sc=False),
    )
    def k(idx_hbm, tab_hbm, out_hbm, idx_v, buf, pooled_v, sem0, sem1):
        wid = lax.axis_index("s") * NC + lax.axis_index("c")
        base = wid * rows_per_w
        sems = (sem0, sem1)
        pltpu.sync_copy(idx_hbm.at[pl.ds(base * S, rows_per_w * S)], idx_v)

        flat = rows_per_w * S

        def remap(g, carry):
            r = idx_v[pl.ds(g * _LANES, _LANES)]
            m = r + r - jnp.where(r >= H, jnp.int32(sub), jnp.int32(0))
            idx_v[pl.ds(g * _LANES, _LANES)] = m
            return carry

        lax.fori_loop(0, flat // _LANES, remap, 0)

        def issue(i, slot, sem):
            for off, c in chunks:
                pltpu.async_copy(
                    tab_hbm.at[idx_v.at[pl.ds(i * S + off, c)]],
                    buf.at[slot, pl.ds(off, c)],
                    sem,
                )

        def wait(i, slot, sem):
            for off, c in chunks:
                pltpu.make_async_copy(
                    tab_hbm.at[idx_v.at[pl.ds(i * S + off, c)]],
                    buf.at[slot, pl.ds(off, c)],
                    sem,
                ).wait()

        issue(0, 0, sems[0])

        def outer(t, carry):
            for s in range(nbuf):
                i = t * nbuf + s
                ns = (s + 1) % nbuf

                @pl.when(i + 1 < rows_per_w)
                def _():
                    issue(i + 1, ns, sems[ns])

                wait(i, s, sems[s])

                accs = tuple(
                    buf[s, 0, pl.ds(d * _LANES, _LANES)] for d in range(nd)
                )
                for r in (1, 2, 3):
                    accs = tuple(
                        jnp.maximum(a, buf[s, r, pl.ds(d * _LANES, _LANES)])
                        for d, a in enumerate(accs)
                    )

                def red4(kk, accs, s=s):
                    jbase = kk * 4
                    for r in range(4):
                        accs = tuple(
                            jnp.maximum(
                                a, buf[s, jbase + r, pl.ds(d * _LANES, _LANES)]
                            )
                            for d, a in enumerate(accs)
                        )
                    return accs

                accs = lax.fori_loop(1, S // 4, red4, accs)
                for d, a in enumerate(accs):
                    pooled_v[i, pl.ds(d * _LANES, _LANES)] = a
            return carry

        lax.fori_loop(0, rows_per_w // nbuf, outer, 0)
        pltpu.sync_copy(pooled_v, out_hbm.at[pl.ds(base, rows_per_w)])

    return k(indices_flat, tab_lin)


def _linear_tc(pooled, W, b):
    B, D = pooled.shape
    O = W.shape[0]
    blk = 512

    def mm(p_ref, w_ref, b_ref, o_ref):
        o_ref[...] = (
            lax.dot_general(
                p_ref[...],
                w_ref[...],
                dimension_numbers=(((1,), (1,)), ((), ())),
                preferred_element_type=jnp.float32,
            )
            + b_ref[...]
        )

    return pl.pallas_call(
        mm,
        grid=(B // blk,),
        in_specs=[
            pl.BlockSpec((blk, D), lambda i: (i, 0)),
            pl.BlockSpec((O, D), lambda i: (0, 0)),
            pl.BlockSpec((1, O), lambda i: (0, 0)),
        ],
        out_specs=pl.BlockSpec((blk, O), lambda i: (i, 0)),
        out_shape=jax.ShapeDtypeStruct((B, O), jnp.float32),
    )(pooled, W, b.reshape(1, O))


def kernel(indices, emb_table, W, b):
    V, D = emb_table.shape
    H = 507904
    assert V == 1000000 and D == 64
    P = _transpose_pack(emb_table.T, V, D, H)
    tab_lin = P.reshape(2 * H, D)
    B, S = indices.shape
    pooled = _pooled_sc(indices.reshape(B * S), tab_lin, H, B, S)
    return _linear_tc(pooled, W, b)


# speedup vs baseline: 7.5017x; 1.0256x over previous
import functools

import jax
import jax.numpy as jnp
from jax import lax
from jax.experimental import pallas as pl
from jax.experimental.pallas import tpu as pltpu
from jax.experimental.pallas import tpu_sc as plsc

_LANES = 16


def _seq_chunks(S):
    chunks = []
    off = 0
    while off < S:
        c = min(128, S - off)
        chunks.append((off, c))
        off += c
    return chunks


def _transpose_pack(tabT, V, D, H):
    TR = 128
    BK = 16384
    SUB = BK // TR
    eye = jnp.eye(TR, dtype=jnp.float32)

    def body(e_ref, x1_ref, x2_ref, o_ref):
        e = e_ref[...]
        dn = (((1,), (1,)), ((), ()))
        for m in range(SUB):
            sl = pl.ds(m * TR, TR)
            t1 = lax.dot_general(
                e, x1_ref[:, sl], dn, preferred_element_type=jnp.float32
            )
            t2 = lax.dot_general(
                e, x2_ref[:, sl], dn, preferred_element_type=jnp.float32
            )
            o_ref[sl, :] = jnp.concatenate([t1, t2], axis=1)

    max_blk = (V - 1) // BK

    return pl.pallas_call(
        body,
        grid=(H // BK,),
        in_specs=[
            pl.BlockSpec((TR, TR), lambda i: (0, 0)),
            pl.BlockSpec((D, BK), lambda i: (0, i)),
            pl.BlockSpec((D, BK), lambda i: (0, jnp.minimum(i + H // BK, max_blk))),
        ],
        out_specs=pl.BlockSpec((BK, 2 * D), lambda i: (i, 0)),
        out_shape=jax.ShapeDtypeStruct((H, 2 * D), jnp.float32),
    )(eye, tabT, tabT)


def _pooled_sc(indices_flat, tab_lin, H, B, S):
    V2, D = tab_lin.shape
    info = plsc.get_sparse_core_info()
    NC, NS = info.num_cores, info.num_subcores
    NW = NC * NS
    assert B % NW == 0
    rows_per_w = B // NW
    chunks = _seq_chunks(S)
    nd = D // _LANES
    assert (rows_per_w * S) % _LANES == 0 and S % 4 == 0
    nbuf = 2
    assert rows_per_w % nbuf == 0
    sub = 2 * H - 1

    mesh = plsc.VectorSubcoreMesh(core_axis_name="c", subcore_axis_name="s")

    @functools.partial(
        pl.kernel,
        mesh=mesh,
        out_type=jax.ShapeDtypeStruct((B, D), jnp.float32),
        scratch_types=[
            pltpu.VMEM((rows_per_w * S,), jnp.int32),
            pltpu.VMEM((nbuf, S, D), jnp.float32),
            pltpu.VMEM((rows_per_w, D), jnp.float32),
            pltpu.SemaphoreType.DMA,
            pltpu.SemaphoreType.DMA,
        ],
        compiler_params=pltpu.CompilerParams(use_tc_tiling_on_sc=False),
    )
    def k(idx_hbm, tab_hbm, out_hbm, idx_v, buf, pooled_v, sem0, sem1):
        wid = lax.axis_index("s") * NC + lax.axis_index("c")
        base = wid * rows_per_w
        sems = (sem0, sem1)
        pltpu.sync_copy(idx_hbm.at[pl.ds(base * S, rows_per_w * S)], idx_v)

        flat = rows_per_w * S

        def remap(g, carry):
            r = idx_v[pl.ds(g * _LANES, _LANES)]
            m = r + r - jnp.where(r >= H, jnp.int32(sub), jnp.int32(0))
            idx_v[pl.ds(g * _LANES, _LANES)] = m
            return carry

        lax.fori_loop(0, flat // _LANES, remap, 0)

        def issue(i, slot, sem):
            for off, c in chunks:
                pltpu.async_copy(
                    tab_hbm.at[idx_v.at[pl.ds(i * S + off, c)]],
                    buf.at[slot, pl.ds(off, c)],
                    sem,
                )

        def wait(i, slot, sem):
            for off, c in chunks:
                pltpu.make_async_copy(
                    tab_hbm.at[idx_v.at[pl.ds(i * S + off, c)]],
                    buf.at[slot, pl.ds(off, c)],
                    sem,
                ).wait()

        issue(0, 0, sems[0])

        def outer(t, carry):
            for s in range(nbuf):
                i = t * nbuf + s
                ns = (s + 1) % nbuf

                @pl.when(i + 1 < rows_per_w)
                def _():
                    issue(i + 1, ns, sems[ns])

                wait(i, s, sems[s])

                accs = tuple(
                    buf[s, 0, pl.ds(d * _LANES, _LANES)] for d in range(nd)
                )
                for r in (1, 2, 3):
                    accs = tuple(
                        jnp.maximum(a, buf[s, r, pl.ds(d * _LANES, _LANES)])
                        for d, a in enumerate(accs)
                    )

                def red4(kk, accs, s=s):
                    jbase = kk * 4
                    for r in range(4):
                        accs = tuple(
                            jnp.maximum(
                                a, buf[s, jbase + r, pl.ds(d * _LANES, _LANES)]
                            )
                            for d, a in enumerate(accs)
                        )
                    return accs

                accs = lax.fori_loop(1, S // 4, red4, accs)
                for d, a in enumerate(accs):
                    pooled_v[i, pl.ds(d * _LANES, _LANES)] = a
            return carry

        lax.fori_loop(0, rows_per_w // nbuf, outer, 0)
        pltpu.sync_copy(pooled_v, out_hbm.at[pl.ds(base, rows_per_w)])

    return k(indices_flat, tab_lin)


def _linear_tc(pooled, W, b):
    B, D = pooled.shape
    O = W.shape[0]
    blk = 512

    def mm(p_ref, w_ref, b_ref, o_ref):
        o_ref[...] = (
            lax.dot_general(
                p_ref[...],
                w_ref[...],
                dimension_numbers=(((1,), (1,)), ((), ())),
                preferred_element_type=jnp.float32,
            )
            + b_ref[...]
        )

    return pl.pallas_call(
        mm,
        grid=(B // blk,),
        in_specs=[
            pl.BlockSpec((blk, D), lambda i: (i, 0)),
            pl.BlockSpec((O, D), lambda i: (0, 0)),
            pl.BlockSpec((1, O), lambda i: (0, 0)),
        ],
        out_specs=pl.BlockSpec((blk, O), lambda i: (i, 0)),
        out_shape=jax.ShapeDtypeStruct((B, O), jnp.float32),
    )(pooled, W, b.reshape(1, O))


def kernel(indices, emb_table, W, b):
    V, D = emb_table.shape
    H = 507904
    assert V == 1000000 and D == 64
    P = _transpose_pack(emb_table.T, V, D, H)
    tab_lin = P.reshape(2 * H, D)
    B, S = indices.shape
    pooled = _pooled_sc(indices.reshape(B * S), tab_lin, H, B, S)
    return _linear_tc(pooled, W, b)


# speedup vs baseline: 8.3017x; 1.1066x over previous
import functools

import jax
import jax.numpy as jnp
from jax import lax
from jax.experimental import pallas as pl
from jax.experimental.pallas import tpu as pltpu
from jax.experimental.pallas import tpu_sc as plsc

_LANES = 16


def _seq_chunks(S):
    chunks = []
    off = 0
    while off < S:
        c = min(128, S - off)
        chunks.append((off, c))
        off += c
    return chunks


def _transpose_pack(tabT, V, D, H):
    TR = 128
    BK = 16384
    SUB = BK // TR
    eye = jnp.eye(TR, dtype=jnp.float32)

    def body(e_ref, x1_ref, x2_ref, o_ref):
        e = e_ref[...]
        dn = (((1,), (1,)), ((), ()))
        for m in range(SUB):
            sl = pl.ds(m * TR, TR)
            t1 = lax.dot_general(
                e, x1_ref[:, sl], dn, preferred_element_type=jnp.float32
            )
            t2 = lax.dot_general(
                e, x2_ref[:, sl], dn, preferred_element_type=jnp.float32
            )
            o_ref[sl, :] = jnp.concatenate([t1, t2], axis=1)

    max_blk = (V - 1) // BK

    return pl.pallas_call(
        body,
        grid=(H // BK,),
        in_specs=[
            pl.BlockSpec((TR, TR), lambda i: (0, 0)),
            pl.BlockSpec((D, BK), lambda i: (0, i)),
            pl.BlockSpec((D, BK), lambda i: (0, jnp.minimum(i + H // BK, max_blk))),
        ],
        out_specs=pl.BlockSpec((BK, 2 * D), lambda i: (i, 0)),
        out_shape=jax.ShapeDtypeStruct((H, 2 * D), jnp.float32),
    )(eye, tabT, tabT)


def _pooled_sc(indices_flat, tab_lin, H, B, S):
    V2, D = tab_lin.shape
    info = plsc.get_sparse_core_info()
    NC, NS = info.num_cores, info.num_subcores
    NW = NC * NS
    assert B % NW == 0
    rows_per_w = B // NW
    chunks = _seq_chunks(S)
    nd = D // _LANES
    assert (rows_per_w * S) % _LANES == 0 and S % 4 == 0
    nbuf = 4
    assert rows_per_w % nbuf == 0
    sub = 2 * H - 1

    mesh = plsc.VectorSubcoreMesh(core_axis_name="c", subcore_axis_name="s")

    @functools.partial(
        pl.kernel,
        mesh=mesh,
        out_type=jax.ShapeDtypeStruct((B, D), jnp.float32),
        scratch_types=[
            pltpu.VMEM((rows_per_w * S,), jnp.int32),
            pltpu.VMEM((nbuf, S, D), jnp.float32),
            pltpu.VMEM((rows_per_w, D), jnp.float32),
            pltpu.SemaphoreType.DMA,
            pltpu.SemaphoreType.DMA,
            pltpu.SemaphoreType.DMA,
            pltpu.SemaphoreType.DMA,
        ],
        compiler_params=pltpu.CompilerParams(use_tc_tiling_on_sc=False),
    )
    def k(idx_hbm, tab_hbm, out_hbm, idx_v, buf, pooled_v, sem0, sem1, sem2, sem3):
        wid = lax.axis_index("s") * NC + lax.axis_index("c")
        base = wid * rows_per_w
        sems = (sem0, sem1, sem2, sem3)
        pltpu.sync_copy(idx_hbm.at[pl.ds(base * S, rows_per_w * S)], idx_v)

        flat = rows_per_w * S

        def remap(g, carry):
            r = idx_v[pl.ds(g * _LANES, _LANES)]
            m = r + r - jnp.where(r >= H, jnp.int32(sub), jnp.int32(0))
            idx_v[pl.ds(g * _LANES, _LANES)] = m
            return carry

        lax.fori_loop(0, flat // _LANES, remap, 0)

        def issue(i, slot, sem):
            for off, c in chunks:
                pltpu.async_copy(
                    tab_hbm.at[idx_v.at[pl.ds(i * S + off, c)]],
                    buf.at[slot, pl.ds(off, c)],
                    sem,
                )

        def wait(i, slot, sem):
            for off, c in chunks:
                pltpu.make_async_copy(
                    tab_hbm.at[idx_v.at[pl.ds(i * S + off, c)]],
                    buf.at[slot, pl.ds(off, c)],
                    sem,
                ).wait()

        issue(0, 0, sems[0])
        issue(1, 1, sems[1])

        def outer(t, carry):
            for s in range(nbuf):
                i = t * nbuf + s
                ns = (s + 2) % nbuf

                @pl.when(i + 2 < rows_per_w)
                def _():
                    issue(i + 2, ns, sems[ns])

                wait(i, s, sems[s])

                accs = tuple(
                    buf[s, 0, pl.ds(d * _LANES, _LANES)] for d in range(nd)
                )
                for r in (1, 2, 3):
                    accs = tuple(
                        jnp.maximum(a, buf[s, r, pl.ds(d * _LANES, _LANES)])
                        for d, a in enumerate(accs)
                    )

                def red4(kk, accs, s=s):
                    jbase = kk * 4
                    for r in range(4):
                        accs = tuple(
                            jnp.maximum(
                                a, buf[s, jbase + r, pl.ds(d * _LANES, _LANES)]
                            )
                            for d, a in enumerate(accs)
                        )
                    return accs

                accs = lax.fori_loop(1, S // 4, red4, accs)
                for d, a in enumerate(accs):
                    pooled_v[i, pl.ds(d * _LANES, _LANES)] = a
            return carry

        lax.fori_loop(0, rows_per_w // nbuf, outer, 0)
        pltpu.sync_copy(pooled_v, out_hbm.at[pl.ds(base, rows_per_w)])

    return k(indices_flat, tab_lin)


def _linear_tc(pooled, W, b):
    B, D = pooled.shape
    O = W.shape[0]
    blk = 512

    def mm(p_ref, w_ref, b_ref, o_ref):
        o_ref[...] = (
            lax.dot_general(
                p_ref[...],
                w_ref[...],
                dimension_numbers=(((1,), (1,)), ((), ())),
                preferred_element_type=jnp.float32,
            )
            + b_ref[...]
        )

    return pl.pallas_call(
        mm,
        grid=(B // blk,),
        in_specs=[
            pl.BlockSpec((blk, D), lambda i: (i, 0)),
            pl.BlockSpec((O, D), lambda i: (0, 0)),
            pl.BlockSpec((1, O), lambda i: (0, 0)),
        ],
        out_specs=pl.BlockSpec((blk, O), lambda i: (i, 0)),
        out_shape=jax.ShapeDtypeStruct((B, O), jnp.float32),
    )(pooled, W, b.reshape(1, O))


def kernel(indices, emb_table, W, b):
    V, D = emb_table.shape
    H = 507904
    assert V == 1000000 and D == 64
    P = _transpose_pack(emb_table.T, V, D, H)
    tab_lin = P.reshape(2 * H, D)
    B, S = indices.shape
    pooled = _pooled_sc(indices.reshape(B * S), tab_lin, H, B, S)
    return _linear_tc(pooled, W, b)


# speedup vs baseline: 8.5816x; 1.0337x over previous
import functools

import jax
import jax.numpy as jnp
from jax import lax
from jax.experimental import pallas as pl
from jax.experimental.pallas import tpu as pltpu
from jax.experimental.pallas import tpu_sc as plsc

_LANES = 16


def _seq_chunks(S):
    chunks = []
    off = 0
    while off < S:
        c = min(128, S - off)
        chunks.append((off, c))
        off += c
    return chunks


def _transpose_pack(tabT, V, D, H):
    TR = 128
    BK = 16384
    SUB = BK // TR
    eye = jnp.eye(TR, dtype=jnp.float32)

    def body(e_ref, x1_ref, x2_ref, o_ref):
        e = e_ref[...]
        dn = (((1,), (1,)), ((), ()))
        for m in range(SUB):
            sl = pl.ds(m * TR, TR)
            t1 = lax.dot_general(
                e, x1_ref[:, sl], dn, preferred_element_type=jnp.float32
            )
            t2 = lax.dot_general(
                e, x2_ref[:, sl], dn, preferred_element_type=jnp.float32
            )
            o_ref[sl, :] = jnp.concatenate([t1, t2], axis=1)

    max_blk = (V - 1) // BK

    return pl.pallas_call(
        body,
        grid=(H // BK,),
        in_specs=[
            pl.BlockSpec((TR, TR), lambda i: (0, 0)),
            pl.BlockSpec((D, BK), lambda i: (0, i)),
            pl.BlockSpec((D, BK), lambda i: (0, jnp.minimum(i + H // BK, max_blk))),
        ],
        out_specs=pl.BlockSpec((BK, 2 * D), lambda i: (i, 0)),
        out_shape=jax.ShapeDtypeStruct((H, 2 * D), jnp.float32),
    )(eye, tabT, tabT)


def _pooled_sc(indices_flat, tab_lin, H, B, S):
    V2, D = tab_lin.shape
    info = plsc.get_sparse_core_info()
    NC, NS = info.num_cores, info.num_subcores
    NW = NC * NS
    assert B % NW == 0
    rows_per_w = B // NW
    chunks = _seq_chunks(S)
    nd = D // _LANES
    assert (rows_per_w * S) % _LANES == 0 and S % 8 == 0
    nbuf = 4
    assert rows_per_w % nbuf == 0
    sub = 2 * H - 1

    mesh = plsc.VectorSubcoreMesh(core_axis_name="c", subcore_axis_name="s")

    @functools.partial(
        pl.kernel,
        mesh=mesh,
        out_type=jax.ShapeDtypeStruct((B, D), jnp.float32),
        scratch_types=[
            pltpu.VMEM((rows_per_w * S,), jnp.int32),
            pltpu.VMEM((nbuf, S, D), jnp.float32),
            pltpu.VMEM((rows_per_w, D), jnp.float32),
            pltpu.SemaphoreType.DMA,
            pltpu.SemaphoreType.DMA,
            pltpu.SemaphoreType.DMA,
            pltpu.SemaphoreType.DMA,
        ],
        compiler_params=pltpu.CompilerParams(use_tc_tiling_on_sc=False),
    )
    def k(idx_hbm, tab_hbm, out_hbm, idx_v, buf, pooled_v, sem0, sem1, sem2, sem3):
        wid = lax.axis_index("s") * NC + lax.axis_index("c")
        base = wid * rows_per_w
        sems = (sem0, sem1, sem2, sem3)
        pltpu.sync_copy(idx_hbm.at[pl.ds(base * S, rows_per_w * S)], idx_v)

        flat = rows_per_w * S

        def remap(g, carry):
            r = idx_v[pl.ds(g * _LANES, _LANES)]
            m = r + r - jnp.where(r >= H, jnp.int32(sub), jnp.int32(0))
            idx_v[pl.ds(g * _LANES, _LANES)] = m
            return carry

        lax.fori_loop(0, flat // _LANES, remap, 0)

        def issue(i, slot, sem):
            for off, c in chunks:
                pltpu.async_copy(
                    tab_hbm.at[idx_v.at[pl.ds(i * S + off, c)]],
                    buf.at[slot, pl.ds(off, c)],
                    sem,
                )

        def wait(i, slot, sem):
            for off, c in chunks:
                pltpu.make_async_copy(
                    tab_hbm.at[idx_v.at[pl.ds(i * S + off, c)]],
                    buf.at[slot, pl.ds(off, c)],
                    sem,
                ).wait()

        issue(0, 0, sems[0])
        issue(1, 1, sems[1])
        issue(2, 2, sems[2])

        def outer(t, carry):
            for s in range(nbuf):
                i = t * nbuf + s
                ns = (s + 3) % nbuf

                @pl.when(i + 3 < rows_per_w)
                def _():
                    issue(i + 3, ns, sems[ns])

                wait(i, s, sems[s])

                accs = tuple(
                    buf[s, 0, pl.ds(d * _LANES, _LANES)] for d in range(nd)
                )
                for r in (1, 2, 3):
                    accs = tuple(
                        jnp.maximum(a, buf[s, r, pl.ds(d * _LANES, _LANES)])
                        for d, a in enumerate(accs)
                    )

                for r in (4, 5, 6, 7):
                    accs = tuple(
                        jnp.maximum(a, buf[s, r, pl.ds(d * _LANES, _LANES)])
                        for d, a in enumerate(accs)
                    )

                def red4(kk, accs, s=s):
                    jbase = kk * 8
                    for r in range(8):
                        accs = tuple(
                            jnp.maximum(
                                a, buf[s, jbase + r, pl.ds(d * _LANES, _LANES)]
                            )
                            for d, a in enumerate(accs)
                        )
                    return accs

                accs = lax.fori_loop(1, S // 8, red4, accs)
                for d, a in enumerate(accs):
                    pooled_v[i, pl.ds(d * _LANES, _LANES)] = a
            return carry

        lax.fori_loop(0, rows_per_w // nbuf, outer, 0)
        pltpu.sync_copy(pooled_v, out_hbm.at[pl.ds(base, rows_per_w)])

    return k(indices_flat, tab_lin)


def _linear_tc(pooled, W, b):
    B, D = pooled.shape
    O = W.shape[0]
    blk = 512

    def mm(p_ref, w_ref, b_ref, o_ref):
        o_ref[...] = (
            lax.dot_general(
                p_ref[...],
                w_ref[...],
                dimension_numbers=(((1,), (1,)), ((), ())),
                preferred_element_type=jnp.float32,
            )
            + b_ref[...]
        )

    return pl.pallas_call(
        mm,
        grid=(B // blk,),
        in_specs=[
            pl.BlockSpec((blk, D), lambda i: (i, 0)),
            pl.BlockSpec((O, D), lambda i: (0, 0)),
            pl.BlockSpec((1, O), lambda i: (0, 0)),
        ],
        out_specs=pl.BlockSpec((blk, O), lambda i: (i, 0)),
        out_shape=jax.ShapeDtypeStruct((B, O), jnp.float32),
    )(pooled, W, b.reshape(1, O))


def kernel(indices, emb_table, W, b):
    V, D = emb_table.shape
    H = 507904
    assert V == 1000000 and D == 64
    P = _transpose_pack(emb_table.T, V, D, H)
    tab_lin = P.reshape(2 * H, D)
    B, S = indices.shape
    pooled = _pooled_sc(indices.reshape(B * S), tab_lin, H, B, S)
    return _linear_tc(pooled, W, b)
